# baseline (speedup 1.0000x reference)
import functools

import jax
import jax.numpy as jnp
from jax import lax
from jax.experimental import pallas as pl
from jax.experimental.pallas import tpu as pltpu
from jax.experimental.pallas import tpu_sc as plsc

DIM = 2048
NUM_EXPERTS = 64
NUM_GROUPS = 8
GROUP_SIZE = 8
NUM_LIMITED_GROUPS = 4
TOP_K = 8
ROUTE_SCALE = 2.5
NEG_INF = float("-inf")

T = 16384
NW = 32
TPW = T // NW
L = 16
CH = 128
NCH = TPW // CH
CSTEPS = CH // L


def _gate_body(x_ref, w_ref, b_ref, sb_ref):
    logits = jax.lax.dot_general(
        w_ref[...], x_ref[...], (((1,), (1,)), ((), ())),
        preferred_element_type=jnp.float32)
    biased = jax.nn.sigmoid(logits) + b_ref[...]
    for k in range(sb_ref.shape[0]):
        sb_ref[k, :, :] = biased[:, k * TPW:(k + 1) * TPW]


def _gate_scores(x, gate_w, expert_bias):
    wpb = 4
    return pl.pallas_call(
        _gate_body,
        grid=(NW // wpb,),
        in_specs=[
            pl.BlockSpec((wpb * TPW, DIM), lambda i: (i, 0)),
            pl.BlockSpec((NUM_EXPERTS, DIM), lambda i: (0, 0)),
            pl.BlockSpec((NUM_EXPERTS, 1), lambda i: (0, 0)),
        ],
        out_specs=pl.BlockSpec((wpb, NUM_EXPERTS, TPW), lambda i: (i, 0, 0)),
        out_shape=jax.ShapeDtypeStruct((NW, NUM_EXPERTS, TPW), jnp.float32),
    )(x, gate_w, expert_bias.reshape(NUM_EXPERTS, 1))


def _tourney(pairs):
    while len(pairs) > 1:
        nxt = []
        for i in range(0, len(pairs) - 1, 2):
            (vl, il), (vr, ir) = pairs[i], pairs[i + 1]
            cond = vl >= vr
            nxt.append((jnp.where(cond, vl, vr), jnp.where(cond, il, ir)))
        if len(pairs) % 2:
            nxt.append(pairs[-1])
        pairs = nxt
    return pairs[0]


def _sc_router_body(biased_hbm, bias_hbm, outv_hbm, outi_hbm, hist_hbm,
                    bblk, bias_v, outv, outi, cnts, cloc):
    wid = lax.axis_index("s") * 2 + lax.axis_index("c")

    lanes = lax.iota(jnp.int32, L)
    zeros = jnp.zeros((L,), jnp.float32)
    ones = jnp.ones((L,), jnp.float32)
    izeros = jnp.zeros((L,), jnp.int32)
    ninf = jnp.full((L,), NEG_INF, jnp.float32)
    sixteen = jnp.full((L,), L, jnp.int32)
    lanes64 = lanes * jnp.full((L,), NUM_EXPERTS, jnp.int32)
    for c in range(L * NUM_EXPERTS // L):
        cnts[pl.ds(c * L, L)] = zeros

    pltpu.sync_copy(bias_hbm, bias_v)

    def step(i, cols):
        sl = pl.ds(i * L, L)

        gsum = []
        for g in range(NUM_GROUPS):
            a = b = None
            for j in range(GROUP_SIZE):
                v = bblk[g * GROUP_SIZE + j, sl]
                if a is None:
                    a, b = v, ninf
                else:
                    b = jnp.maximum(b, jnp.minimum(a, v))
                    a = jnp.maximum(a, v)
            gsum.append(a + b)

        keep = []
        for g in range(NUM_GROUPS):
            rank = None
            for h in range(NUM_GROUPS):
                if h == g:
                    continue
                beats = (gsum[h] >= gsum[g]) if h < g else (gsum[h] > gsum[g])
                r = beats.astype(jnp.int32)
                rank = r if rank is None else rank + r
            keep.append(rank < NUM_LIMITED_GROUPS)

        keepi = [jnp.where(k, jnp.full((L,), 1, jnp.int32), izeros)
                 for k in keep]
        pos = [izeros]
        for g in range(1, NUM_GROUPS):
            pos.append(pos[g - 1] + keepi[g - 1])
        gbase = []
        for s in range(NUM_LIMITED_GROUPS):
            s_const = jnp.full((L,), s, jnp.int32)
            acc = izeros
            for g in range(NUM_GROUPS):
                hit = keep[g] & (pos[g] == s_const)
                acc = acc + jnp.where(hit,
                                      jnp.full((L,), g * GROUP_SIZE, jnp.int32),
                                      izeros)
            gbase.append(acc)

        cand = []
        for s in range(NUM_LIMITED_GROUPS):
            for j in range(GROUP_SIZE):
                cidx = gbase[s] + jnp.full((L,), j, jnp.int32)
                cval = plsc.load_gather(bblk, [cidx, cols])
                cand.append((cval, cidx))

        svals, sidxs = [], []
        for _r in range(TOP_K):
            win_v, win_i = _tourney(list(cand))
            sval = win_v - plsc.load_gather(bias_v, [win_i])
            plsc.addupdate_scatter(cnts, [lanes64 + win_i], ones)
            svals.append(sval)
            sidxs.append(win_i)
            cand = [(jnp.where(ci == win_i, ninf, cv), ci)
                    for (cv, ci) in cand]

        ssum = svals[0]
        for r in range(1, TOP_K):
            ssum = ssum + svals[r]
        scale = ROUTE_SCALE / (ssum + 1e-20)
        for r in range(TOP_K):
            rcol = jnp.full((L,), r, jnp.int32)
            plsc.store_scatter(outv, [cols, rcol], svals[r] * scale)
            plsc.store_scatter(outi, [cols, rcol], sidxs[r])
        return cols + sixteen

    base = wid * TPW

    def chunk(c, carry):
        pltpu.sync_copy(biased_hbm.at[wid, :, pl.ds(c * CH, CH)], bblk)
        lax.fori_loop(0, CSTEPS, step, lanes)
        pltpu.sync_copy(outv, outv_hbm.at[pl.ds(base + c * CH, CH)])
        pltpu.sync_copy(outi, outi_hbm.at[pl.ds(base + c * CH, CH)])
        return carry

    lax.fori_loop(0, NCH, chunk, 0)

    for c in range(NUM_EXPERTS // L):
        acc = cnts[pl.ds(c * L, L)]
        for l in range(1, L):
            acc = acc + cnts[pl.ds(l * NUM_EXPERTS + c * L, L)]
        cloc[pl.ds(c * L, L)] = acc

    pltpu.sync_copy(cloc, hist_hbm.at[wid])


@functools.partial(
    pl.kernel,
    mesh=plsc.VectorSubcoreMesh(core_axis_name="c", subcore_axis_name="s"),
    compiler_params=pltpu.CompilerParams(needs_layout_passes=False),
    out_type=[
        jax.ShapeDtypeStruct((T, TOP_K), jnp.float32),
        jax.ShapeDtypeStruct((T, TOP_K), jnp.int32),
        jax.ShapeDtypeStruct((NW, NUM_EXPERTS), jnp.float32),
    ],
    scratch_types=[
        pltpu.VMEM((NUM_EXPERTS, CH), jnp.float32),
        pltpu.VMEM((NUM_EXPERTS,), jnp.float32),
        pltpu.VMEM((CH, TOP_K), jnp.float32),
        pltpu.VMEM((CH, TOP_K), jnp.int32),
        pltpu.VMEM((L * NUM_EXPERTS,), jnp.float32),
        pltpu.VMEM((NUM_EXPERTS,), jnp.float32),
    ],
)
def _sc_router(biased_hbm, bias_hbm, outv_hbm, outi_hbm, hist_hbm,
               bblk, bias_v, outv, outi, cnts, cloc):
    _sc_router_body(biased_hbm, bias_hbm, outv_hbm, outi_hbm, hist_hbm,
                    bblk, bias_v, outv, outi, cnts, cloc)


def kernel(x, expert_bias, gate_w):
    biased3 = _gate_scores(x, gate_w, expert_bias)
    tv, ti, histp = _sc_router(biased3, expert_bias)
    return tv, ti, histp.sum(axis=0)


# speedup vs baseline: 1.0075x; 1.0075x over previous
import functools

import jax
import jax.numpy as jnp
from jax import lax
from jax.experimental import pallas as pl
from jax.experimental.pallas import tpu as pltpu
from jax.experimental.pallas import tpu_sc as plsc

DIM = 2048
NUM_EXPERTS = 64
NUM_GROUPS = 8
GROUP_SIZE = 8
NUM_LIMITED_GROUPS = 4
TOP_K = 8
ROUTE_SCALE = 2.5
NEG_INF = float("-inf")

T = 16384
NW = 32
TPW = T // NW
L = 16
CH = 128
NCH = TPW // CH
CSTEPS = CH // L


def _gate_body(x_ref, w_ref, b_ref, sb_ref):
    logits = jax.lax.dot_general(
        w_ref[...], x_ref[...], (((1,), (1,)), ((), ())),
        preferred_element_type=jnp.float32)
    biased = jax.nn.sigmoid(logits) + b_ref[...]
    for k in range(sb_ref.shape[0]):
        sb_ref[k, :, :] = biased[:, k * TPW:(k + 1) * TPW]


def _gate_scores(x, gate_w, expert_bias):
    wpb = 2
    return pl.pallas_call(
        _gate_body,
        grid=(NW // wpb,),
        in_specs=[
            pl.BlockSpec((wpb * TPW, DIM), lambda i: (i, 0)),
            pl.BlockSpec((NUM_EXPERTS, DIM), lambda i: (0, 0)),
            pl.BlockSpec((NUM_EXPERTS, 1), lambda i: (0, 0)),
        ],
        out_specs=pl.BlockSpec((wpb, NUM_EXPERTS, TPW), lambda i: (i, 0, 0)),
        out_shape=jax.ShapeDtypeStruct((NW, NUM_EXPERTS, TPW), jnp.float32),
    )(x, gate_w, expert_bias.reshape(NUM_EXPERTS, 1))


def _tourney(pairs):
    while len(pairs) > 1:
        nxt = []
        for i in range(0, len(pairs) - 1, 2):
            (vl, il), (vr, ir) = pairs[i], pairs[i + 1]
            cond = vl >= vr
            nxt.append((jnp.where(cond, vl, vr), jnp.where(cond, il, ir)))
        if len(pairs) % 2:
            nxt.append(pairs[-1])
        pairs = nxt
    return pairs[0]


def _sc_router_body(biased_hbm, bias_hbm, outv_hbm, outi_hbm, hist_hbm,
                    bblk, bias_v, outv, outi, cnts, cloc):
    wid = lax.axis_index("s") * 2 + lax.axis_index("c")

    lanes = lax.iota(jnp.int32, L)
    zeros = jnp.zeros((L,), jnp.float32)
    ones = jnp.ones((L,), jnp.float32)
    izeros = jnp.zeros((L,), jnp.int32)
    ninf = jnp.full((L,), NEG_INF, jnp.float32)
    sixteen = jnp.full((L,), L, jnp.int32)
    lanes64 = lanes * jnp.full((L,), NUM_EXPERTS, jnp.int32)
    for c in range(L * NUM_EXPERTS // L):
        cnts[pl.ds(c * L, L)] = zeros

    pltpu.sync_copy(bias_hbm, bias_v)

    def step(i, cols):
        sl = pl.ds(i * L, L)

        gsum = []
        for g in range(NUM_GROUPS):
            a = b = None
            for j in range(GROUP_SIZE):
                v = bblk[g * GROUP_SIZE + j, sl]
                if a is None:
                    a, b = v, ninf
                else:
                    b = jnp.maximum(b, jnp.minimum(a, v))
                    a = jnp.maximum(a, v)
            gsum.append(a + b)

        keep = []
        for g in range(NUM_GROUPS):
            rank = None
            for h in range(NUM_GROUPS):
                if h == g:
                    continue
                beats = (gsum[h] >= gsum[g]) if h < g else (gsum[h] > gsum[g])
                r = beats.astype(jnp.int32)
                rank = r if rank is None else rank + r
            keep.append(rank < NUM_LIMITED_GROUPS)

        keepi = [jnp.where(k, jnp.full((L,), 1, jnp.int32), izeros)
                 for k in keep]
        pos = [izeros]
        for g in range(1, NUM_GROUPS):
            pos.append(pos[g - 1] + keepi[g - 1])
        gbase = []
        for s in range(NUM_LIMITED_GROUPS):
            s_const = jnp.full((L,), s, jnp.int32)
            acc = izeros
            for g in range(NUM_GROUPS):
                hit = keep[g] & (pos[g] == s_const)
                acc = acc + jnp.where(hit,
                                      jnp.full((L,), g * GROUP_SIZE, jnp.int32),
                                      izeros)
            gbase.append(acc)

        cand = []
        for s in range(NUM_LIMITED_GROUPS):
            for j in range(GROUP_SIZE):
                cidx = gbase[s] + jnp.full((L,), j, jnp.int32)
                cval = plsc.load_gather(bblk, [cidx, cols])
                cand.append((cval, cidx))

        svals, sidxs = [], []
        for _r in range(TOP_K):
            win_v, win_i = _tourney(list(cand))
            sval = win_v - plsc.load_gather(bias_v, [win_i])
            plsc.addupdate_scatter(cnts, [lanes64 + win_i], ones)
            svals.append(sval)
            sidxs.append(win_i)
            cand = [(jnp.where(ci == win_i, ninf, cv), ci)
                    for (cv, ci) in cand]

        ssum = svals[0]
        for r in range(1, TOP_K):
            ssum = ssum + svals[r]
        scale = ROUTE_SCALE / (ssum + 1e-20)
        for r in range(TOP_K):
            rcol = jnp.full((L,), r, jnp.int32)
            plsc.store_scatter(outv, [cols, rcol], svals[r] * scale)
            plsc.store_scatter(outi, [cols, rcol], sidxs[r])
        return cols + sixteen

    base = wid * TPW

    def chunk(c, carry):
        pltpu.sync_copy(biased_hbm.at[wid, :, pl.ds(c * CH, CH)], bblk)
        lax.fori_loop(0, CSTEPS, step, lanes)
        pltpu.sync_copy(outv, outv_hbm.at[pl.ds(base + c * CH, CH)])
        pltpu.sync_copy(outi, outi_hbm.at[pl.ds(base + c * CH, CH)])
        return carry

    lax.fori_loop(0, NCH, chunk, 0)

    for c in range(NUM_EXPERTS // L):
        acc = cnts[pl.ds(c * L, L)]
        for l in range(1, L):
            acc = acc + cnts[pl.ds(l * NUM_EXPERTS + c * L, L)]
        cloc[pl.ds(c * L, L)] = acc

    pltpu.sync_copy(cloc, hist_hbm.at[wid])


@functools.partial(
    pl.kernel,
    mesh=plsc.VectorSubcoreMesh(core_axis_name="c", subcore_axis_name="s"),
    compiler_params=pltpu.CompilerParams(needs_layout_passes=False),
    out_type=[
        jax.ShapeDtypeStruct((T, TOP_K), jnp.float32),
        jax.ShapeDtypeStruct((T, TOP_K), jnp.int32),
        jax.ShapeDtypeStruct((NW, NUM_EXPERTS), jnp.float32),
    ],
    scratch_types=[
        pltpu.VMEM((NUM_EXPERTS, CH), jnp.float32),
        pltpu.VMEM((NUM_EXPERTS,), jnp.float32),
        pltpu.VMEM((CH, TOP_K), jnp.float32),
        pltpu.VMEM((CH, TOP_K), jnp.int32),
        pltpu.VMEM((L * NUM_EXPERTS,), jnp.float32),
        pltpu.VMEM((NUM_EXPERTS,), jnp.float32),
    ],
)
def _sc_router(biased_hbm, bias_hbm, outv_hbm, outi_hbm, hist_hbm,
               bblk, bias_v, outv, outi, cnts, cloc):
    _sc_router_body(biased_hbm, bias_hbm, outv_hbm, outi_hbm, hist_hbm,
                    bblk, bias_v, outv, outi, cnts, cloc)


def kernel(x, expert_bias, gate_w):
    biased3 = _gate_scores(x, gate_w, expert_bias)
    tv, ti, histp = _sc_router(biased3, expert_bias)
    return tv, ti, histp.sum(axis=0)
